# baseline (speedup 1.0000x reference)
import functools

import jax
import jax.numpy as jnp
from jax import lax
from jax.experimental import pallas as pl
from jax.experimental.pallas import tpu as pltpu
from jax.experimental.pallas import tpu_sc as plsc

NP = 1000000
D = 64
B = 16384

NC = 2
NS = 16
NW = NC * NS
CHUNK = 512
NFULL = NP // CHUNK
TAIL_LO = NFULL * CHUNK
TAIL_W = NP - TAIL_LO
CAP = 640
NVEC = B // 16
OUT1_W = NW * CAP
JCH = CAP // 128


def _pass1_body(tblT, idx_hbm, out1, colbuf, colj, bstart,
                sem_i, sem_c0, sem_c1, sem_o):
    wid = lax.axis_index("s") * NC + lax.axis_index("c")
    is_w0 = wid == 0
    is_wlast = wid == NW - 1
    cstart = jnp.where(is_w0, 0, wid * 61 + 1)
    nchw = jnp.where(is_w0, 62, 61)
    lo_w = cstart * CHUNK
    hi_w = jnp.where(is_wlast, NP, (cstart + nchw) * CHUNK)
    iota = lax.iota(jnp.int32, 16)

    def scoped(cand_p, cand_j, bkt_p, bkt_j):
        for k in range(CAP // 16):
            cand_p[pl.ds(16 * k, 16)] = jnp.full((16,), -1, jnp.int32)
            bkt_j[pl.ds(16 * k, 16)] = B + iota + 16 * k

        def scope_idx(idxv):
            pltpu.async_copy(idx_hbm, idxv, sem_i).wait()

            def scan(k, off):
                v = idxv[pl.ds(16 * k, 16)]
                m = (v >= lo_w) & (v < hi_w)
                plsc.store_compressed(cand_p.at[pl.ds(off, 16)], v, mask=m)
                plsc.store_compressed(cand_j.at[pl.ds(off, 16)],
                                      iota + 16 * k, mask=m)
                return off + plsc.all_reduce_population_count(m)[0]

            bstart[63] = lax.fori_loop(0, NVEC, scan, 0)

        pl.run_scoped(scope_idx, pltpu.VMEM((B,), jnp.int32))
        count = bstart[63]
        nvec_c = (count + 15) // 16
        nb = nchw + jnp.where(is_wlast, 1, 0)

        def scope_stream(chunk, chunkT):
            def fire(i):
                lo_i = pl.multiple_of((cstart + i) * CHUNK, 128)
                src = tblT.at[:, pl.ds(lo_i, CHUNK)]

                @pl.when(i % 2 == 0)
                def _():
                    pltpu.make_async_copy(src, chunk.at[0], sem_c0).start()

                @pl.when(i % 2 == 1)
                def _():
                    pltpu.make_async_copy(src, chunk.at[1], sem_c1).start()

            fire(0)

            @pl.when(1 < nchw)
            def _():
                fire(1)

            def bucket(i, boff):
                blo = (cstart + i) * CHUNK
                bhi = jnp.minimum(blo + CHUNK, NP)
                bstart[i] = boff

                def one(g, o):
                    pv = cand_p[pl.ds(16 * g, 16)]
                    m = (pv >= blo) & (pv < bhi)
                    jv = cand_j[pl.ds(16 * g, 16)]
                    plsc.store_compressed(bkt_p.at[pl.ds(o, 16)], pv, mask=m)
                    plsc.store_compressed(bkt_j.at[pl.ds(o, 16)], jv, mask=m)
                    return o + plsc.all_reduce_population_count(m)[0]

                return lax.fori_loop(0, nvec_c, one, boff)

            total = lax.fori_loop(0, nb, bucket, 0)
            bstart[nb] = total

            def gather_from(chunk_ref, lo_i, s0, s1):
                ngrp = (s1 - s0 + 15) // 16

                def grp(g, carry):
                    base = s0 + 16 * g
                    pv = bkt_p[pl.ds(base, 16)]
                    m = iota < (s1 - base)
                    lv = pv - lo_i
                    slot = iota + base
                    for d in range(D):
                        dfull = jnp.full((16,), d, jnp.int32)
                        vals = plsc.load_gather(chunk_ref, [dfull, lv],
                                                mask=m)
                        plsc.store_scatter(colbuf, [dfull, slot], vals,
                                           mask=m)
                    return carry

                lax.fori_loop(0, ngrp, grp, 0)

            def run_chunk(i, carry):
                lo_i = pl.multiple_of((cstart + i) * CHUNK, 128)

                @pl.when(i % 2 == 0)
                def _():
                    pltpu.make_async_copy(tblT.at[:, pl.ds(0, CHUNK)],
                                          chunk.at[0], sem_c0).wait()
                    gather_from(chunk.at[0], lo_i, bstart[i], bstart[i + 1])

                @pl.when(i % 2 == 1)
                def _():
                    pltpu.make_async_copy(tblT.at[:, pl.ds(0, CHUNK)],
                                          chunk.at[1], sem_c1).wait()
                    gather_from(chunk.at[1], lo_i, bstart[i], bstart[i + 1])

                @pl.when(i + 2 < nchw)
                def _():
                    fire(i + 2)

                return carry

            lax.fori_loop(0, nchw, run_chunk, 0)

            @pl.when(is_wlast)
            def _tail():
                pltpu.async_copy(tblT.at[:, pl.ds(TAIL_LO, TAIL_W)], chunkT,
                                 sem_c0).wait()
                gather_from(chunkT, TAIL_LO, bstart[61], bstart[62])

        pl.run_scoped(scope_stream,
                      pltpu.VMEM((2, D, CHUNK), jnp.float32),
                      pltpu.VMEM((D, TAIL_W), jnp.float32))

        for k in range(CAP // 16):
            jv = bkt_j[pl.ds(16 * k, 16)]
            colj[0, pl.ds(16 * k, 16)] = plsc.bitcast(jv, jnp.float32)

    pl.run_scoped(scoped,
                  pltpu.VMEM((CAP,), jnp.int32),
                  pltpu.VMEM((CAP,), jnp.int32),
                  pltpu.VMEM((CAP,), jnp.int32),
                  pltpu.VMEM((CAP,), jnp.int32))

    wbase = pl.multiple_of(wid * CAP, 128)
    pltpu.async_copy(colbuf, out1.at[pl.ds(0, D), pl.ds(wbase, CAP)],
                     sem_o).wait()
    pltpu.async_copy(colj, out1.at[pl.ds(D, 8), pl.ds(wbase, CAP)],
                     sem_o).wait()


def _pass2_body(out1_hbm, out_hbm, colblk, rowblk, jmapv, sem_in, sem_s):
    wid = lax.axis_index("s") * NC + lax.axis_index("c")
    wbase = pl.multiple_of(wid * CAP, 128)
    iota = lax.iota(jnp.int32, 16)
    pltpu.async_copy(out1_hbm.at[:, pl.ds(wbase, CAP)], colblk, sem_in).wait()
    for k in range(JCH):
        for m in range(8):
            v = colblk[D, pl.ds(128 * k + 16 * m, 16)]
            jmapv[k, pl.ds(16 * m, 16)] = plsc.bitcast(v, jnp.int32)
    for k in range(JCH):
        def trans_d(d, carry, _k=k):
            dfull = jnp.full((16,), d, jnp.int32)
            for m in range(8):
                lane = iota + (128 * _k + 16 * m)
                vals = plsc.load_gather(colblk, [dfull, lane])
                plsc.store_scatter(rowblk, [iota + 16 * m, dfull], vals)
            return carry

        lax.fori_loop(0, D, trans_d, 0)
        pltpu.async_copy(rowblk, out_hbm.at[jmapv.at[k]], sem_s).wait()


@jax.jit
def _gather(patterns, idx):
    mesh = plsc.VectorSubcoreMesh(core_axis_name="c", subcore_axis_name="s")
    pass1 = functools.partial(
        pl.kernel,
        mesh=mesh,
        out_type=jax.ShapeDtypeStruct((D + 8, OUT1_W), jnp.float32),
        scratch_types=[
            pltpu.VMEM((D, CAP), jnp.float32),
            pltpu.VMEM((8, CAP), jnp.float32),
            pltpu.SMEM((64,), jnp.int32),
            pltpu.SemaphoreType.DMA,
            pltpu.SemaphoreType.DMA,
            pltpu.SemaphoreType.DMA,
            pltpu.SemaphoreType.DMA,
        ],
        compiler_params=pltpu.CompilerParams(use_tc_tiling_on_sc=True,
                                             needs_layout_passes=False),
    )(_pass1_body)
    out1 = pass1(patterns.T, idx)

    pass2 = functools.partial(
        pl.kernel,
        mesh=mesh,
        out_type=jax.ShapeDtypeStruct((B + CAP, 128), jnp.float32),
        scratch_types=[
            pltpu.VMEM((D + 8, CAP), jnp.float32),
            pltpu.VMEM((128, 128), jnp.float32),
            pltpu.VMEM((JCH, 128), jnp.int32),
            pltpu.SemaphoreType.DMA,
            pltpu.SemaphoreType.DMA,
        ],
        compiler_params=pltpu.CompilerParams(use_tc_tiling_on_sc=True,
                                             needs_layout_passes=False),
    )(_pass2_body)
    out = pass2(out1)
    return out[:B, :D]


def kernel(x, patterns):
    idx = jax.random.randint(jax.random.key(42), (x.shape[0],), 0,
                             patterns.shape[0], dtype=jnp.int32)
    return _gather(patterns, idx)


# speedup vs baseline: 1.2284x; 1.2284x over previous
import functools

import jax
import jax.numpy as jnp
from jax import lax
from jax.experimental import pallas as pl
from jax.experimental.pallas import tpu as pltpu
from jax.experimental.pallas import tpu_sc as plsc

NP = 1000000
D = 64
B = 16384

NC = 2
NS = 16
NW = NC * NS
CHUNK = 512
NFULL = NP // CHUNK
TAIL_LO = NFULL * CHUNK
TAIL_W = NP - TAIL_LO
CAP = 640
NVEC = B // 16
NBATCH = CAP // 128


def _body(tblT, idx_hbm, out_hbm, jmapv, bstart,
          sem_i, sem_c0, sem_c1, sem_s):
    wid = lax.axis_index("s") * NC + lax.axis_index("c")
    is_w0 = wid == 0
    is_wlast = wid == NW - 1
    cstart = jnp.where(is_w0, 0, wid * 61 + 1)
    nchw = jnp.where(is_w0, 62, 61)
    lo_w = cstart * CHUNK
    hi_w = jnp.where(is_wlast, NP, (cstart + nchw) * CHUNK)
    iota = lax.iota(jnp.int32, 16)

    def scoped(cand_p, cand_j, bkt_p, bkt_j, rowring):
        for k in range(CAP // 16):
            cand_p[pl.ds(16 * k, 16)] = jnp.full((16,), -1, jnp.int32)
            bkt_j[pl.ds(16 * k, 16)] = B + iota + 16 * k

        def scope_idx(idxv):
            pltpu.async_copy(idx_hbm, idxv, sem_i).wait()

            def scan(k, off):
                v = idxv[pl.ds(16 * k, 16)]
                m = (v >= lo_w) & (v < hi_w)
                plsc.store_compressed(cand_p.at[pl.ds(off, 16)], v, mask=m)
                plsc.store_compressed(cand_j.at[pl.ds(off, 16)],
                                      iota + 16 * k, mask=m)
                return off + plsc.all_reduce_population_count(m)[0]

            bstart[63] = lax.fori_loop(0, NVEC, scan, 0)

        pl.run_scoped(scope_idx, pltpu.VMEM((B,), jnp.int32))
        count = bstart[63]
        nvec_c = (count + 15) // 16
        nb = nchw + jnp.where(is_wlast, 1, 0)

        def scope_stream(chunk, chunkT):
            def fire(i):
                lo_i = pl.multiple_of((cstart + i) * CHUNK, 128)
                src = tblT.at[:, pl.ds(lo_i, CHUNK)]

                @pl.when(i % 2 == 0)
                def _():
                    pltpu.make_async_copy(src, chunk.at[0], sem_c0).start()

                @pl.when(i % 2 == 1)
                def _():
                    pltpu.make_async_copy(src, chunk.at[1], sem_c1).start()

            fire(0)

            @pl.when(1 < nchw)
            def _():
                fire(1)

            def bucket(i, boff):
                blo = (cstart + i) * CHUNK
                bhi = jnp.minimum(blo + CHUNK, NP)
                bstart[i] = boff

                def one(g, o):
                    pv = cand_p[pl.ds(16 * g, 16)]
                    m = (pv >= blo) & (pv < bhi)
                    jv = cand_j[pl.ds(16 * g, 16)]
                    plsc.store_compressed(bkt_p.at[pl.ds(o, 16)], pv, mask=m)
                    plsc.store_compressed(bkt_j.at[pl.ds(o, 16)], jv, mask=m)
                    return o + plsc.all_reduce_population_count(m)[0]

                return lax.fori_loop(0, nvec_c, one, boff)

            total = lax.fori_loop(0, nb, bucket, 0)
            bstart[nb] = total
            for k in range(NBATCH):
                for m in range(8):
                    jmapv[k, pl.ds(16 * m, 16)] = bkt_j[
                        pl.ds(128 * k + 16 * m, 16)]

            def gather_from(chunk_ref, lo_i, s0, s1):
                ngrp = (s1 - s0 + 15) // 16

                def grp(g, carry):
                    base = s0 + 16 * g
                    pv = bkt_p[pl.ds(base, 16)]
                    m = iota < (s1 - base)
                    lv = pv - lo_i
                    rr = (iota + base) % 256
                    for d in range(D):
                        dfull = jnp.full((16,), d, jnp.int32)
                        vals = plsc.load_gather(chunk_ref, [dfull, lv],
                                                mask=m)
                        plsc.store_scatter(rowring, [rr, dfull], vals,
                                           mask=m)
                    return carry

                lax.fori_loop(0, ngrp, grp, 0)

            def flush_batch(k):
                pltpu.async_copy(
                    rowring.at[pl.ds((k % 2) * 128, 128)],
                    out_hbm.at[jmapv.at[k]], sem_s).wait()

            def run_chunk(i, carry):
                lo_i = pl.multiple_of((cstart + i) * CHUNK, 128)

                @pl.when(i % 2 == 0)
                def _():
                    pltpu.make_async_copy(tblT.at[:, pl.ds(0, CHUNK)],
                                          chunk.at[0], sem_c0).wait()
                    gather_from(chunk.at[0], lo_i, bstart[i], bstart[i + 1])

                @pl.when(i % 2 == 1)
                def _():
                    pltpu.make_async_copy(tblT.at[:, pl.ds(0, CHUNK)],
                                          chunk.at[1], sem_c1).wait()
                    gather_from(chunk.at[1], lo_i, bstart[i], bstart[i + 1])

                @pl.when(i + 2 < nchw)
                def _():
                    fire(i + 2)

                for k in range(NBATCH):
                    @pl.when(((k + 1) * 128 <= bstart[i + 1])
                             & ((k + 1) * 128 > bstart[i]))
                    def _(k=k):
                        flush_batch(k)

                return carry

            lax.fori_loop(0, nchw, run_chunk, 0)
            filled_main = bstart[nchw]

            @pl.when(is_wlast)
            def _tail():
                pltpu.async_copy(tblT.at[:, pl.ds(TAIL_LO, TAIL_W)], chunkT,
                                 sem_c0).wait()
                gather_from(chunkT, TAIL_LO, bstart[61], bstart[62])

            for k in range(NBATCH):
                @pl.when((k + 1) * 128 > filled_main)
                def _(k=k):
                    flush_batch(k)

        pl.run_scoped(scope_stream,
                      pltpu.VMEM((2, D, CHUNK), jnp.float32),
                      pltpu.VMEM((D, TAIL_W), jnp.float32))

    pl.run_scoped(scoped,
                  pltpu.VMEM((CAP,), jnp.int32),
                  pltpu.VMEM((CAP,), jnp.int32),
                  pltpu.VMEM((CAP,), jnp.int32),
                  pltpu.VMEM((CAP,), jnp.int32),
                  pltpu.VMEM((256, 128), jnp.float32))


@jax.jit
def _gather(patterns, idx):
    mesh = plsc.VectorSubcoreMesh(core_axis_name="c", subcore_axis_name="s")
    run = functools.partial(
        pl.kernel,
        mesh=mesh,
        out_type=jax.ShapeDtypeStruct((B + CAP, 128), jnp.float32),
        scratch_types=[
            pltpu.VMEM((NBATCH, 128), jnp.int32),
            pltpu.SMEM((64,), jnp.int32),
            pltpu.SemaphoreType.DMA,
            pltpu.SemaphoreType.DMA,
            pltpu.SemaphoreType.DMA,
            pltpu.SemaphoreType.DMA,
        ],
        compiler_params=pltpu.CompilerParams(use_tc_tiling_on_sc=True,
                                             needs_layout_passes=False),
    )(_body)
    out = run(patterns.T, idx)
    return out[:B, :D]


def kernel(x, patterns):
    idx = jax.random.randint(jax.random.key(42), (x.shape[0],), 0,
                             patterns.shape[0], dtype=jnp.int32)
    return _gather(patterns, idx)


# speedup vs baseline: 1.2437x; 1.0124x over previous
import functools

import jax
import jax.numpy as jnp
from jax import lax
from jax.experimental import pallas as pl
from jax.experimental.pallas import tpu as pltpu
from jax.experimental.pallas import tpu_sc as plsc

NP = 1000000
D = 64
B = 16384

NC = 2
NS = 16
NW = NC * NS
CHUNK = 512
NFULL = NP // CHUNK
TAIL_LO = NFULL * CHUNK
TAIL_W = NP - TAIL_LO
CAP = 640
NVEC = B // 16
NBATCH = CAP // 128


def _body(tblT, idx_hbm, out_hbm, jmapv, bstart,
          sem_i, sem_c0, sem_c1, sem_s):
    wid = lax.axis_index("s") * NC + lax.axis_index("c")
    is_w0 = wid == 0
    is_wlast = wid == NW - 1
    cstart = jnp.where(is_w0, 0, wid * 61 + 1)
    nchw = jnp.where(is_w0, 62, 61)
    lo_w = cstart * CHUNK
    hi_w = jnp.where(is_wlast, NP, (cstart + nchw) * CHUNK)
    iota = lax.iota(jnp.int32, 16)

    def scoped(cand_p, cand_j, crs_p, crs_j, bkt_p, bkt_j, rowring,
               idxv, chunk, chunkT):
        def fire(i):
            lo_i = pl.multiple_of((cstart + i) * CHUNK, 128)
            src = tblT.at[:, pl.ds(lo_i, CHUNK)]

            @pl.when(i % 2 == 0)
            def _():
                pltpu.make_async_copy(src, chunk.at[0], sem_c0).start()

            @pl.when(i % 2 == 1)
            def _():
                pltpu.make_async_copy(src, chunk.at[1], sem_c1).start()

        pltpu.make_async_copy(idx_hbm, idxv, sem_i).start()
        fire(0)

        @pl.when(1 < nchw)
        def _():
            fire(1)

        for k in range(CAP // 16):
            cand_p[pl.ds(16 * k, 16)] = jnp.full((16,), -1, jnp.int32)
            crs_p[pl.ds(16 * k, 16)] = jnp.full((16,), -1, jnp.int32)
            bkt_j[pl.ds(16 * k, 16)] = B + iota + 16 * k

        pltpu.make_async_copy(idx_hbm, idxv, sem_i).wait()

        def scan(k, off):
            v = idxv[pl.ds(16 * k, 16)]
            m = (v >= lo_w) & (v < hi_w)
            plsc.store_compressed(cand_p.at[pl.ds(off, 16)], v, mask=m)
            plsc.store_compressed(cand_j.at[pl.ds(off, 16)],
                                  iota + 16 * k, mask=m)
            return off + plsc.all_reduce_population_count(m)[0]

        count = lax.fori_loop(0, NVEC, scan, 0)
        nvec_c = (count + 15) // 16
        nb = nchw + jnp.where(is_wlast, 1, 0)

        coff = 0
        for r in range(8):
            bstart[64 + r] = coff
            rlo = (cstart + 8 * r) * CHUNK
            rhi = jnp.minimum(
                (cstart + jnp.minimum(8 * (r + 1), nb)) * CHUNK, NP)

            def one_c(g, o, _rlo=rlo, _rhi=rhi):
                pv = cand_p[pl.ds(16 * g, 16)]
                m = (pv >= _rlo) & (pv < _rhi)
                jv = cand_j[pl.ds(16 * g, 16)]
                plsc.store_compressed(crs_p.at[pl.ds(o, 16)], pv, mask=m)
                plsc.store_compressed(crs_j.at[pl.ds(o, 16)], jv, mask=m)
                return o + plsc.all_reduce_population_count(m)[0]

            coff = lax.fori_loop(0, nvec_c, one_c, coff)
        bstart[72] = coff

        def bucket(i, boff):
            blo = (cstart + i) * CHUNK
            bhi = jnp.minimum(blo + CHUNK, NP)
            bstart[i] = boff
            r = i // 8
            cs0 = bstart[64 + r]
            ng = (bstart[64 + r + 1] - cs0 + 15) // 16

            def one(g, o):
                pv = crs_p[pl.ds(cs0 + 16 * g, 16)]
                m = (pv >= blo) & (pv < bhi)
                jv = crs_j[pl.ds(cs0 + 16 * g, 16)]
                plsc.store_compressed(bkt_p.at[pl.ds(o, 16)], pv, mask=m)
                plsc.store_compressed(bkt_j.at[pl.ds(o, 16)], jv, mask=m)
                return o + plsc.all_reduce_population_count(m)[0]

            return lax.fori_loop(0, ng, one, boff)

        total = lax.fori_loop(0, nb, bucket, 0)
        bstart[nb] = total
        for k in range(NBATCH):
            for m in range(8):
                jmapv[k, pl.ds(16 * m, 16)] = bkt_j[
                    pl.ds(128 * k + 16 * m, 16)]

        if True:

            def gather_from(chunk_ref, lo_i, s0, s1):
                ngrp = (s1 - s0 + 15) // 16

                def grp(g, carry):
                    base = s0 + 16 * g
                    pv = bkt_p[pl.ds(base, 16)]
                    m = iota < (s1 - base)
                    lv = pv - lo_i
                    rr = (iota + base) % 256
                    for d in range(D):
                        dfull = jnp.full((16,), d, jnp.int32)
                        vals = plsc.load_gather(chunk_ref, [dfull, lv],
                                                mask=m)
                        plsc.store_scatter(rowring, [rr, dfull], vals,
                                           mask=m)
                    return carry

                lax.fori_loop(0, ngrp, grp, 0)

            def flush_batch(k):
                pltpu.async_copy(
                    rowring.at[pl.ds((k % 2) * 128, 128)],
                    out_hbm.at[jmapv.at[k]], sem_s).wait()

            def run_chunk(i, carry):
                lo_i = pl.multiple_of((cstart + i) * CHUNK, 128)

                @pl.when(i % 2 == 0)
                def _():
                    pltpu.make_async_copy(tblT.at[:, pl.ds(0, CHUNK)],
                                          chunk.at[0], sem_c0).wait()
                    gather_from(chunk.at[0], lo_i, bstart[i], bstart[i + 1])

                @pl.when(i % 2 == 1)
                def _():
                    pltpu.make_async_copy(tblT.at[:, pl.ds(0, CHUNK)],
                                          chunk.at[1], sem_c1).wait()
                    gather_from(chunk.at[1], lo_i, bstart[i], bstart[i + 1])

                @pl.when(i + 2 < nchw)
                def _():
                    fire(i + 2)

                for k in range(NBATCH):
                    @pl.when(((k + 1) * 128 <= bstart[i + 1])
                             & ((k + 1) * 128 > bstart[i]))
                    def _(k=k):
                        flush_batch(k)

                return carry

            lax.fori_loop(0, nchw, run_chunk, 0)
            filled_main = bstart[nchw]

            @pl.when(is_wlast)
            def _tail():
                pltpu.async_copy(tblT.at[:, pl.ds(TAIL_LO, TAIL_W)], chunkT,
                                 sem_c0).wait()
                gather_from(chunkT, TAIL_LO, bstart[61], bstart[62])

            for k in range(NBATCH):
                @pl.when((k + 1) * 128 > filled_main)
                def _(k=k):
                    flush_batch(k)

    pl.run_scoped(scoped,
                  pltpu.VMEM((CAP,), jnp.int32),
                  pltpu.VMEM((CAP,), jnp.int32),
                  pltpu.VMEM((CAP,), jnp.int32),
                  pltpu.VMEM((CAP,), jnp.int32),
                  pltpu.VMEM((CAP,), jnp.int32),
                  pltpu.VMEM((CAP,), jnp.int32),
                  pltpu.VMEM((256, 128), jnp.float32),
                  pltpu.VMEM((B,), jnp.int32),
                  pltpu.VMEM((2, D, CHUNK), jnp.float32),
                  pltpu.VMEM((D, TAIL_W), jnp.float32))


@jax.jit
def _gather(patterns, idx):
    mesh = plsc.VectorSubcoreMesh(core_axis_name="c", subcore_axis_name="s")
    run = functools.partial(
        pl.kernel,
        mesh=mesh,
        out_type=jax.ShapeDtypeStruct((B + CAP, 128), jnp.float32),
        scratch_types=[
            pltpu.VMEM((NBATCH, 128), jnp.int32),
            pltpu.SMEM((80,), jnp.int32),
            pltpu.SemaphoreType.DMA,
            pltpu.SemaphoreType.DMA,
            pltpu.SemaphoreType.DMA,
            pltpu.SemaphoreType.DMA,
        ],
        compiler_params=pltpu.CompilerParams(use_tc_tiling_on_sc=True,
                                             needs_layout_passes=False),
    )(_body)
    out = run(patterns.T, idx)
    return out[:B, :D]


def kernel(x, patterns):
    idx = jax.random.randint(jax.random.key(42), (x.shape[0],), 0,
                             patterns.shape[0], dtype=jnp.int32)
    return _gather(patterns, idx)


# speedup vs baseline: 1.2730x; 1.0235x over previous
import functools

import jax
import jax.numpy as jnp
from jax import lax
from jax.experimental import pallas as pl
from jax.experimental.pallas import tpu as pltpu
from jax.experimental.pallas import tpu_sc as plsc

NP = 1000000
D = 64
B = 16384

NC = 2
NS = 16
NW = NC * NS
CHUNK = 256
NBUF = 4
NFULL = NP // CHUNK
NCH = 122
TAIL_LO = NFULL * CHUNK
TAIL_W = NP - TAIL_LO
CAP = 640
NVEC = B // 16
NBATCH = CAP // 128


def _body(tblT, idx_hbm, out_hbm, jmapv, bstart,
          sem_i, sem_c0, sem_c1, sem_c2, sem_c3, sem_s):
    wid = lax.axis_index("s") * NC + lax.axis_index("c")
    is_w0 = wid == 0
    is_wlast = wid == NW - 1
    cstart = wid * NCH + jnp.minimum(wid, 2)
    nchw = jnp.where(wid < 2, NCH + 1, NCH)
    lo_w = cstart * CHUNK
    hi_w = jnp.where(is_wlast, NP, (cstart + nchw) * CHUNK)
    iota = lax.iota(jnp.int32, 16)

    def scoped(cand_p, cand_j, crs_p, crs_j, bkt_p, bkt_j, rowring,
               idxv, chunk, chunkT):
        csems = [sem_c0, sem_c1, sem_c2, sem_c3]

        def fire(i):
            lo_i = pl.multiple_of((cstart + i) * CHUNK, 128)
            src = tblT.at[:, pl.ds(lo_i, CHUNK)]
            for b in range(NBUF):
                @pl.when(i % NBUF == b)
                def _(b=b):
                    pltpu.make_async_copy(src, chunk.at[b], csems[b]).start()

        pltpu.make_async_copy(idx_hbm, idxv, sem_i).start()
        fire(0)
        fire(1)
        fire(2)

        for k in range(CAP // 16):
            cand_p[pl.ds(16 * k, 16)] = jnp.full((16,), -1, jnp.int32)
            crs_p[pl.ds(16 * k, 16)] = jnp.full((16,), -1, jnp.int32)
            bkt_j[pl.ds(16 * k, 16)] = B + iota + 16 * k

        pltpu.make_async_copy(idx_hbm, idxv, sem_i).wait()

        def scan(k, off):
            v = idxv[pl.ds(16 * k, 16)]
            m = (v >= lo_w) & (v < hi_w)
            plsc.store_compressed(cand_p.at[pl.ds(off, 16)], v, mask=m)
            plsc.store_compressed(cand_j.at[pl.ds(off, 16)],
                                  iota + 16 * k, mask=m)
            return off + plsc.all_reduce_population_count(m)[0]

        count = lax.fori_loop(0, NVEC, scan, 0)
        nvec_c = (count + 15) // 16
        nb = nchw + jnp.where(is_wlast, 1, 0)

        coff = 0
        for r in range(8):
            bstart[128 + r] = coff
            rlo = (cstart + 16 * r) * CHUNK
            rhi = jnp.minimum(
                (cstart + jnp.minimum(16 * (r + 1), nb)) * CHUNK, NP)

            def one_c(g, o, _rlo=rlo, _rhi=rhi):
                pv = cand_p[pl.ds(16 * g, 16)]
                m = (pv >= _rlo) & (pv < _rhi)
                jv = cand_j[pl.ds(16 * g, 16)]
                plsc.store_compressed(crs_p.at[pl.ds(o, 16)], pv, mask=m)
                plsc.store_compressed(crs_j.at[pl.ds(o, 16)], jv, mask=m)
                return o + plsc.all_reduce_population_count(m)[0]

            coff = lax.fori_loop(0, nvec_c, one_c, coff)
        bstart[136] = coff

        def bucket(i, boff):
            blo = (cstart + i) * CHUNK
            bhi = jnp.minimum(blo + CHUNK, NP)
            bstart[i] = boff
            r = i // 16
            cs0 = bstart[128 + r]
            ng = (bstart[128 + r + 1] - cs0 + 15) // 16

            def one(g, o):
                pv = crs_p[pl.ds(cs0 + 16 * g, 16)]
                m = (pv >= blo) & (pv < bhi)
                jv = crs_j[pl.ds(cs0 + 16 * g, 16)]
                plsc.store_compressed(bkt_p.at[pl.ds(o, 16)], pv, mask=m)
                plsc.store_compressed(bkt_j.at[pl.ds(o, 16)], jv, mask=m)
                return o + plsc.all_reduce_population_count(m)[0]

            return lax.fori_loop(0, ng, one, boff)

        total = lax.fori_loop(0, nb, bucket, 0)
        bstart[nb] = total
        for k in range(NBATCH):
            for m in range(8):
                jmapv[k, pl.ds(16 * m, 16)] = bkt_j[
                    pl.ds(128 * k + 16 * m, 16)]

        if True:

            def gather_from(chunk_ref, lo_i, s0, s1):
                ngrp = (s1 - s0 + 15) // 16

                def grp(g, carry):
                    base = s0 + 16 * g
                    pv = bkt_p[pl.ds(base, 16)]
                    m = iota < (s1 - base)
                    lv = pv - lo_i
                    rr = (iota + base) % 256
                    for d in range(D):
                        dfull = jnp.full((16,), d, jnp.int32)
                        vals = plsc.load_gather(chunk_ref, [dfull, lv],
                                                mask=m)
                        plsc.store_scatter(rowring, [rr, dfull], vals,
                                           mask=m)
                    return carry

                lax.fori_loop(0, ngrp, grp, 0)

            def flush_batch(k):
                pltpu.async_copy(
                    rowring.at[pl.ds((k % 2) * 128, 128)],
                    out_hbm.at[jmapv.at[k]], sem_s).wait()

            def run_chunk(i, carry):
                lo_i = pl.multiple_of((cstart + i) * CHUNK, 128)

                @pl.when(i + 3 < nchw)
                def _():
                    fire(i + 3)

                for b in range(NBUF):
                    @pl.when(i % NBUF == b)
                    def _(b=b):
                        pltpu.make_async_copy(tblT.at[:, pl.ds(0, CHUNK)],
                                              chunk.at[b], csems[b]).wait()
                        gather_from(chunk.at[b], lo_i,
                                    bstart[i], bstart[i + 1])

                for k in range(NBATCH):
                    @pl.when(((k + 1) * 128 <= bstart[i + 1])
                             & ((k + 1) * 128 > bstart[i]))
                    def _(k=k):
                        flush_batch(k)

                return carry

            lax.fori_loop(0, nchw, run_chunk, 0)
            filled_main = bstart[nchw]

            @pl.when(is_wlast)
            def _tail():
                pltpu.async_copy(tblT.at[:, pl.ds(TAIL_LO, TAIL_W)], chunkT,
                                 sem_c0).wait()
                gather_from(chunkT, TAIL_LO, bstart[NCH], bstart[NCH + 1])

            for k in range(NBATCH):
                @pl.when((k + 1) * 128 > filled_main)
                def _(k=k):
                    flush_batch(k)

    pl.run_scoped(scoped,
                  pltpu.VMEM((CAP,), jnp.int32),
                  pltpu.VMEM((CAP,), jnp.int32),
                  pltpu.VMEM((CAP,), jnp.int32),
                  pltpu.VMEM((CAP,), jnp.int32),
                  pltpu.VMEM((CAP,), jnp.int32),
                  pltpu.VMEM((CAP,), jnp.int32),
                  pltpu.VMEM((256, 128), jnp.float32),
                  pltpu.VMEM((B,), jnp.int32),
                  pltpu.VMEM((NBUF, D, CHUNK), jnp.float32),
                  pltpu.VMEM((D, TAIL_W), jnp.float32))


@jax.jit
def _gather(patterns, idx):
    mesh = plsc.VectorSubcoreMesh(core_axis_name="c", subcore_axis_name="s")
    run = functools.partial(
        pl.kernel,
        mesh=mesh,
        out_type=jax.ShapeDtypeStruct((B + CAP, 128), jnp.float32),
        scratch_types=[
            pltpu.VMEM((NBATCH, 128), jnp.int32),
            pltpu.SMEM((144,), jnp.int32),
            pltpu.SemaphoreType.DMA,
            pltpu.SemaphoreType.DMA,
            pltpu.SemaphoreType.DMA,
            pltpu.SemaphoreType.DMA,
            pltpu.SemaphoreType.DMA,
            pltpu.SemaphoreType.DMA,
        ],
        compiler_params=pltpu.CompilerParams(use_tc_tiling_on_sc=True,
                                             needs_layout_passes=False),
    )(_body)
    out = run(patterns.T, idx)
    return out[:B, :D]


def kernel(x, patterns):
    idx = jax.random.randint(jax.random.key(42), (x.shape[0],), 0,
                             patterns.shape[0], dtype=jnp.int32)
    return _gather(patterns, idx)


# speedup vs baseline: 1.3578x; 1.0666x over previous
import functools

import jax
import jax.numpy as jnp
from jax import lax
from jax.experimental import pallas as pl
from jax.experimental.pallas import tpu as pltpu
from jax.experimental.pallas import tpu_sc as plsc

NP = 1000000
D = 64
B = 16384

NC = 2
NS = 16
NW = NC * NS
CHUNK = 256
NBUF = 4
NFULL = NP // CHUNK
NCH = 122
TAIL_LO = NFULL * CHUNK
TAIL_W = NP - TAIL_LO
CAP = 640
NVEC = B // 16
NBATCH = CAP // 128


def _body(tblT, idx_hbm, out_hbm, jmapv, bstart,
          sem_i, sem_c0, sem_c1, sem_c2, sem_c3, sem_s):
    wid = lax.axis_index("s") * NC + lax.axis_index("c")
    is_w0 = wid == 0
    is_wlast = wid == NW - 1
    cstart = wid * NCH + jnp.minimum(wid, 2)
    nchw = jnp.where(wid < 2, NCH + 1, NCH)
    lo_w = cstart * CHUNK
    hi_w = jnp.where(is_wlast, NP, (cstart + nchw) * CHUNK)
    iota = lax.iota(jnp.int32, 16)

    def scoped(cand_p, cand_j, crs_p, crs_j, bkt_p, bkt_j, rowring,
               idxv, chunk, chunkT):
        csems = [sem_c0, sem_c1, sem_c2, sem_c3]

        def fire(i):
            lo_i = pl.multiple_of((cstart + i) * CHUNK, 128)
            src = tblT.at[:, pl.ds(lo_i, CHUNK)]
            for b in range(NBUF):
                @pl.when(i % NBUF == b)
                def _(b=b):
                    pltpu.make_async_copy(src, chunk.at[b], csems[b]).start()

        pltpu.make_async_copy(idx_hbm, idxv, sem_i).start()
        fire(0)
        fire(1)
        fire(2)

        for k in range(CAP // 16):
            cand_p[pl.ds(16 * k, 16)] = jnp.full((16,), -1, jnp.int32)
            crs_p[pl.ds(16 * k, 16)] = jnp.full((16,), -1, jnp.int32)
            bkt_j[pl.ds(16 * k, 16)] = B + iota + 16 * k

        pltpu.make_async_copy(idx_hbm, idxv, sem_i).wait()

        def scan(k, off):
            v = idxv[pl.ds(16 * k, 16)]
            m = (v >= lo_w) & (v < hi_w)
            plsc.store_compressed(cand_p.at[pl.ds(off, 16)], v, mask=m)
            plsc.store_compressed(cand_j.at[pl.ds(off, 16)],
                                  iota + 16 * k, mask=m)
            return off + plsc.all_reduce_population_count(m)[0]

        count = lax.fori_loop(0, NVEC, scan, 0)
        nvec_c = (count + 15) // 16
        nb = nchw + jnp.where(is_wlast, 1, 0)

        bstart[140] = 0
        bstart[141] = 0

        def bucket_range(r):
            coff = bstart[140]
            bstart[128 + r] = coff
            rlo = (cstart + 16 * r) * CHUNK
            rhi = jnp.minimum(
                (cstart + jnp.minimum(16 * (r + 1), nb)) * CHUNK, NP)

            def one_c(g, o):
                pv = cand_p[pl.ds(16 * g, 16)]
                m = (pv >= rlo) & (pv < rhi)
                jv = cand_j[pl.ds(16 * g, 16)]
                plsc.store_compressed(crs_p.at[pl.ds(o, 16)], pv, mask=m)
                plsc.store_compressed(crs_j.at[pl.ds(o, 16)], jv, mask=m)
                return o + plsc.all_reduce_population_count(m)[0]

            coff2 = lax.fori_loop(0, nvec_c, one_c, coff)
            bstart[140] = coff2
            cs0 = coff
            ng = (coff2 - cs0 + 15) // 16

            def fine(ib, boff):
                b = 16 * r + ib
                blo = (cstart + b) * CHUNK
                bhi = jnp.minimum(blo + CHUNK, NP)
                bstart[b] = boff

                def one(g, o):
                    pv = crs_p[pl.ds(cs0 + 16 * g, 16)]
                    m = (pv >= blo) & (pv < bhi)
                    jv = crs_j[pl.ds(cs0 + 16 * g, 16)]
                    plsc.store_compressed(bkt_p.at[pl.ds(o, 16)], pv,
                                          mask=m)
                    plsc.store_compressed(bkt_j.at[pl.ds(o, 16)], jv,
                                          mask=m)
                    return o + plsc.all_reduce_population_count(m)[0]

                return lax.fori_loop(0, ng, one, boff)

            nr = jnp.minimum(16, nb - 16 * r)
            total_r = lax.fori_loop(0, nr, fine, bstart[141])
            bstart[141] = total_r
            bstart[jnp.minimum(16 * r + nr, nb)] = total_r

        if True:

            def gather_from(chunk_ref, lo_i, s0, s1):
                ngrp = (s1 - s0 + 15) // 16

                def grp(g, carry):
                    base = s0 + 16 * g
                    pv = bkt_p[pl.ds(base, 16)]
                    m = iota < (s1 - base)
                    lv = pv - lo_i
                    rr = (iota + base) % 256
                    for d in range(D):
                        dfull = jnp.full((16,), d, jnp.int32)
                        vals = plsc.load_gather(chunk_ref, [dfull, lv],
                                                mask=m)
                        plsc.store_scatter(rowring, [rr, dfull], vals,
                                           mask=m)
                    return carry

                lax.fori_loop(0, ngrp, grp, 0)

            def flush_batch(k):
                for m in range(8):
                    jmapv[k, pl.ds(16 * m, 16)] = bkt_j[
                        pl.ds(128 * k + 16 * m, 16)]
                pltpu.async_copy(
                    rowring.at[pl.ds((k % 2) * 128, 128)],
                    out_hbm.at[jmapv.at[k]], sem_s).wait()

            def run_chunk(i, carry):
                lo_i = pl.multiple_of((cstart + i) * CHUNK, 128)

                @pl.when(i % 16 == 0)
                def _():
                    bucket_range(i // 16)

                @pl.when(i + 3 < nchw)
                def _():
                    fire(i + 3)

                for b in range(NBUF):
                    @pl.when(i % NBUF == b)
                    def _(b=b):
                        pltpu.make_async_copy(tblT.at[:, pl.ds(0, CHUNK)],
                                              chunk.at[b], csems[b]).wait()
                        gather_from(chunk.at[b], lo_i,
                                    bstart[i], bstart[i + 1])

                for k in range(NBATCH):
                    @pl.when(((k + 1) * 128 <= bstart[i + 1])
                             & ((k + 1) * 128 > bstart[i]))
                    def _(k=k):
                        flush_batch(k)

                return carry

            lax.fori_loop(0, nchw, run_chunk, 0)
            filled_main = bstart[nchw]

            @pl.when(is_wlast)
            def _tail():
                pltpu.async_copy(tblT.at[:, pl.ds(TAIL_LO, TAIL_W)], chunkT,
                                 sem_c0).wait()
                gather_from(chunkT, TAIL_LO, bstart[NCH], bstart[NCH + 1])

            for k in range(NBATCH):
                @pl.when((k + 1) * 128 > filled_main)
                def _(k=k):
                    flush_batch(k)

    pl.run_scoped(scoped,
                  pltpu.VMEM((CAP,), jnp.int32),
                  pltpu.VMEM((CAP,), jnp.int32),
                  pltpu.VMEM((CAP,), jnp.int32),
                  pltpu.VMEM((CAP,), jnp.int32),
                  pltpu.VMEM((CAP,), jnp.int32),
                  pltpu.VMEM((CAP,), jnp.int32),
                  pltpu.VMEM((256, 128), jnp.float32),
                  pltpu.VMEM((B,), jnp.int32),
                  pltpu.VMEM((NBUF, D, CHUNK), jnp.float32),
                  pltpu.VMEM((D, TAIL_W), jnp.float32))


@jax.jit
def _gather(patterns, idx):
    mesh = plsc.VectorSubcoreMesh(core_axis_name="c", subcore_axis_name="s")
    run = functools.partial(
        pl.kernel,
        mesh=mesh,
        out_type=jax.ShapeDtypeStruct((B + CAP, 128), jnp.float32),
        scratch_types=[
            pltpu.VMEM((NBATCH, 128), jnp.int32),
            pltpu.SMEM((144,), jnp.int32),
            pltpu.SemaphoreType.DMA,
            pltpu.SemaphoreType.DMA,
            pltpu.SemaphoreType.DMA,
            pltpu.SemaphoreType.DMA,
            pltpu.SemaphoreType.DMA,
            pltpu.SemaphoreType.DMA,
        ],
        compiler_params=pltpu.CompilerParams(use_tc_tiling_on_sc=True,
                                             needs_layout_passes=False),
    )(_body)
    out = run(patterns.T, idx)
    return out[:B, :D]


def kernel(x, patterns):
    idx = jax.random.randint(jax.random.key(42), (x.shape[0],), 0,
                             patterns.shape[0], dtype=jnp.int32)
    return _gather(patterns, idx)
